# baseline (speedup 1.0000x reference)
import functools

import jax
import jax.numpy as jnp
from jax import lax
from jax.experimental import pallas as pl
from jax.experimental.pallas import tpu as pltpu
from jax.experimental.pallas import tpu_sc as plsc

S = 2048
D = 1024
E = 64
CHUNK = 128
P2 = S + 8 * E
XROWS = P2 + CHUNK
RB = 256
EPG = 2


def _rank_kernel(u_col_ref, u_row_ref, sp_ref):
    u_r = u_row_ref[...]
    u_c = u_col_ref[...]
    er = lax.broadcasted_iota(jnp.int32, (1, E), 1)
    ec = lax.broadcasted_iota(jnp.int32, (E, 1), 0)

    hist = jnp.sum((u_c == er).astype(jnp.int32), axis=0, keepdims=True)
    padded = ((hist + 7) // 8) * 8
    k64 = lax.broadcasted_iota(jnp.int32, (E, E), 1)
    e64 = lax.broadcasted_iota(jnp.int32, (E, E), 0)
    offc = jnp.sum(jnp.where(k64 < e64, padded, 0), axis=1, keepdims=True)

    aoff = jnp.sum(jnp.where(ec == u_r, offc, 0), axis=0, keepdims=True)
    rank = jnp.zeros((1, S), jnp.int32)
    for kb in range(S // RB):
        u_cb = u_col_ref[pl.ds(kb * RB, RB), :]
        k_idx = kb * RB + lax.broadcasted_iota(jnp.int32, (RB, S), 0)
        j_idx = lax.broadcasted_iota(jnp.int32, (RB, S), 1)
        m = (u_cb == u_r) & (k_idx < j_idx)
        rank = rank + jnp.sum(m.astype(jnp.int32), axis=0, keepdims=True)
    sp_ref[...] = aoff + rank


def _expert_kernel(u_ref, xs_ref, b_ref, w_ref, w2_ref, out_ref):
    i = pl.program_id(0)
    u = u_ref[...]
    b_full = b_ref[...]
    ec = lax.broadcasted_iota(jnp.int32, (E, 1), 0)
    hist = jnp.sum((ec == u).astype(jnp.int32), axis=1, keepdims=True)
    padded = ((hist + 7) // 8) * 8

    for t in range(EPG):
        e = i * EPG + t
        off = jnp.sum(jnp.where(ec < e, padded, 0))
        off = pl.multiple_of(off, 8)
        cnt = jnp.sum((u == e).astype(jnp.int32))
        nch = (cnt + (CHUNK - 1)) // CHUNK
        b = jnp.sum(jnp.where(ec == e, b_full, 0.0), axis=0, keepdims=True)

        H = D // 2

        def body(c, carry):
            s = off + c * CHUNK
            xa = xs_ref[pl.ds(s, CHUNK), :]
            accl = jnp.dot(xa, w_ref[t], preferred_element_type=jnp.float32)
            accr = jnp.dot(xa, w2_ref[t], preferred_element_type=jnp.float32)
            out_ref[pl.ds(s, CHUNK), 0:H] = jnp.tanh(accl + b[:, 0:H])
            out_ref[pl.ds(s, CHUNK), H:D] = jnp.tanh(accr + b[:, H:D])
            return carry

        lax.fori_loop(0, nch, body, 0)


def _sc_scatter_rows(x, sp, n_out):
    n, d = x.shape
    mesh = plsc.VectorSubcoreMesh(core_axis_name="c", subcore_axis_name="s")
    nw = mesh.num_cores * mesh.num_subcores
    per = n // nw

    @functools.partial(
        pl.kernel,
        out_type=jax.ShapeDtypeStruct((n_out, d), jnp.float32),
        mesh=mesh,
        scratch_types=[
            pltpu.VMEM((per,), jnp.int32),
            pltpu.VMEM((per, d), jnp.float32),
            pltpu.SemaphoreType.DMA,
        ],
    )
    def sk(x_hbm, sp_hbm, out_hbm, idx_v, rows_v, sem):
        wid = lax.axis_index("s") * mesh.num_cores + lax.axis_index("c")
        base = wid * per
        pltpu.sync_copy(sp_hbm.at[pl.ds(base, per)], idx_v)
        pltpu.sync_copy(x_hbm.at[pl.ds(base, per)], rows_v)
        pltpu.async_copy(rows_v, out_hbm.at[idx_v], sem).wait()

    return sk(x, sp)


def _sc_row_gather(table, idx, n_out):
    n_idx = idx.shape[0]
    d = table.shape[1]
    mesh = plsc.VectorSubcoreMesh(core_axis_name="c", subcore_axis_name="s")
    nw = mesh.num_cores * mesh.num_subcores
    per = n_idx // nw

    @functools.partial(
        pl.kernel,
        out_type=jax.ShapeDtypeStruct((n_out, d), jnp.float32),
        mesh=mesh,
        scratch_types=[
            pltpu.VMEM((per,), jnp.int32),
            pltpu.VMEM((per, d), jnp.float32),
            pltpu.SemaphoreType.DMA,
        ],
    )
    def gk(table_hbm, idx_hbm, out_hbm, idx_v, rows_v, sem):
        wid = lax.axis_index("s") * mesh.num_cores + lax.axis_index("c")
        base = wid * per
        pltpu.sync_copy(idx_hbm.at[pl.ds(base, per)], idx_v)
        pltpu.async_copy(table_hbm.at[idx_v], rows_v, sem).wait()
        pltpu.sync_copy(rows_v, out_hbm.at[pl.ds(base, per)])

    return gk(table, idx)


def kernel(input, urls, trans, bias):
    x = input.reshape(S, D)
    u = urls.reshape(S).astype(jnp.int32)

    sp_row = pl.pallas_call(
        _rank_kernel,
        out_shape=jax.ShapeDtypeStruct((1, S), jnp.int32),
    )(u.reshape(S, 1), u.reshape(1, S))
    sp = sp_row.reshape(S)

    xs = _sc_scatter_rows(x, sp, XROWS)

    out_sorted = pl.pallas_call(
        _expert_kernel,
        grid=(E // EPG,),
        in_specs=[
            pl.BlockSpec((1, S), lambda i: (0, 0)),
            pl.BlockSpec((XROWS, D), lambda i: (0, 0)),
            pl.BlockSpec((E, D), lambda i: (0, 0)),
            pl.BlockSpec((EPG, D, D // 2), lambda i: (i, 0, 0)),
            pl.BlockSpec((EPG, D, D // 2), lambda i: (i, 0, 1)),
        ],
        out_specs=pl.BlockSpec((XROWS, D), lambda i: (0, 0)),
        out_shape=jax.ShapeDtypeStruct((XROWS, D), jnp.float32),
    )(u.reshape(1, S), xs, bias, trans, trans)

    out = _sc_row_gather(out_sorted, sp, S)
    return out.reshape(input.shape)


# speedup vs baseline: 1.0086x; 1.0086x over previous
import functools

import jax
import jax.numpy as jnp
from jax import lax
from jax.experimental import pallas as pl
from jax.experimental.pallas import tpu as pltpu
from jax.experimental.pallas import tpu_sc as plsc

S = 2048
D = 1024
E = 64
CHUNK = 128
P2 = S + 8 * E
XROWS = P2 + CHUNK
RB = 256
EPG = 2


def _rank_kernel(u_col_ref, u_row_ref, sp_ref):
    u_r = u_row_ref[...]
    u_c = u_col_ref[...]
    er = lax.broadcasted_iota(jnp.int32, (1, E), 1)
    ec = lax.broadcasted_iota(jnp.int32, (E, 1), 0)

    hist = jnp.sum((u_c == er).astype(jnp.int32), axis=0, keepdims=True)
    padded = ((hist + 7) // 8) * 8
    k64 = lax.broadcasted_iota(jnp.int32, (E, E), 1)
    e64 = lax.broadcasted_iota(jnp.int32, (E, E), 0)
    offc = jnp.sum(jnp.where(k64 < e64, padded, 0), axis=1, keepdims=True)

    aoff = jnp.sum(jnp.where(ec == u_r, offc, 0), axis=0, keepdims=True)
    rank = jnp.zeros((1, S), jnp.int32)
    for kb in range(S // RB):
        u_cb = u_col_ref[pl.ds(kb * RB, RB), :]
        k_idx = kb * RB + lax.broadcasted_iota(jnp.int32, (RB, S), 0)
        j_idx = lax.broadcasted_iota(jnp.int32, (RB, S), 1)
        m = (u_cb == u_r) & (k_idx < j_idx)
        rank = rank + jnp.sum(m.astype(jnp.int32), axis=0, keepdims=True)
    sp_ref[...] = aoff + rank


def _expert_kernel(u_ref, xs_ref, b_ref, w_ref, out_ref):
    i = pl.program_id(0)
    u = u_ref[...]
    b_full = b_ref[...]
    ec = lax.broadcasted_iota(jnp.int32, (E, 1), 0)
    hist = jnp.sum((ec == u).astype(jnp.int32), axis=1, keepdims=True)
    padded = ((hist + 7) // 8) * 8

    for t in range(EPG):
        e = i * EPG + t
        off = jnp.sum(jnp.where(ec < e, padded, 0))
        off = pl.multiple_of(off, 8)
        cnt = jnp.sum((u == e).astype(jnp.int32))
        nch = (cnt + (CHUNK - 1)) // CHUNK
        b = jnp.sum(jnp.where(ec == e, b_full, 0.0), axis=0, keepdims=True)

        def body(c, carry):
            s = off + c * CHUNK
            xa = xs_ref[pl.ds(s, CHUNK), :]
            acc = jnp.dot(xa, w_ref[t], preferred_element_type=jnp.float32)
            out_ref[pl.ds(s, CHUNK), :] = jnp.tanh(acc + b)
            return carry

        lax.fori_loop(0, nch, body, 0)


def _sc_scatter_rows(x, sp, n_out):
    n, d = x.shape
    mesh = plsc.VectorSubcoreMesh(core_axis_name="c", subcore_axis_name="s")
    nw = mesh.num_cores * mesh.num_subcores
    per = n // nw

    @functools.partial(
        pl.kernel,
        out_type=jax.ShapeDtypeStruct((n_out, d), jnp.float32),
        mesh=mesh,
        scratch_types=[
            pltpu.VMEM((per,), jnp.int32),
            pltpu.VMEM((per, d), jnp.float32),
            pltpu.SemaphoreType.DMA,
        ],
    )
    def sk(x_hbm, sp_hbm, out_hbm, idx_v, rows_v, sem):
        wid = lax.axis_index("s") * mesh.num_cores + lax.axis_index("c")
        base = wid * per
        pltpu.sync_copy(sp_hbm.at[pl.ds(base, per)], idx_v)
        pltpu.sync_copy(x_hbm.at[pl.ds(base, per)], rows_v)
        pltpu.async_copy(rows_v, out_hbm.at[idx_v], sem).wait()

    return sk(x, sp)


def _sc_row_gather(table, idx, n_out):
    n_idx = idx.shape[0]
    d = table.shape[1]
    mesh = plsc.VectorSubcoreMesh(core_axis_name="c", subcore_axis_name="s")
    nw = mesh.num_cores * mesh.num_subcores
    per = n_idx // nw

    @functools.partial(
        pl.kernel,
        out_type=jax.ShapeDtypeStruct((n_out, d), jnp.float32),
        mesh=mesh,
        scratch_types=[
            pltpu.VMEM((per,), jnp.int32),
            pltpu.VMEM((per, d), jnp.float32),
            pltpu.SemaphoreType.DMA,
        ],
    )
    def gk(table_hbm, idx_hbm, out_hbm, idx_v, rows_v, sem):
        wid = lax.axis_index("s") * mesh.num_cores + lax.axis_index("c")
        base = wid * per
        pltpu.sync_copy(idx_hbm.at[pl.ds(base, per)], idx_v)
        pltpu.async_copy(table_hbm.at[idx_v], rows_v, sem).wait()
        pltpu.sync_copy(rows_v, out_hbm.at[pl.ds(base, per)])

    return gk(table, idx)


def kernel(input, urls, trans, bias):
    x = input.reshape(S, D)
    u = urls.reshape(S).astype(jnp.int32)

    sp_row = pl.pallas_call(
        _rank_kernel,
        out_shape=jax.ShapeDtypeStruct((1, S), jnp.int32),
    )(u.reshape(S, 1), u.reshape(1, S))
    sp = sp_row.reshape(S)

    xs = _sc_scatter_rows(x, sp, XROWS)

    out_sorted = pl.pallas_call(
        _expert_kernel,
        grid=(E // EPG,),
        in_specs=[
            pl.BlockSpec((1, S), lambda i: (0, 0)),
            pl.BlockSpec((XROWS, D), lambda i: (0, 0)),
            pl.BlockSpec((E, D), lambda i: (0, 0)),
            pl.BlockSpec((EPG, D, D), lambda i: (i, 0, 0)),
        ],
        out_specs=pl.BlockSpec((XROWS, D), lambda i: (0, 0)),
        out_shape=jax.ShapeDtypeStruct((XROWS, D), jnp.float32),
    )(u.reshape(1, S), xs, bias, trans)

    out = _sc_row_gather(out_sorted, sp, S)
    return out.reshape(input.shape)
